# Initial kernel scaffold; baseline (speedup 1.0000x reference)
import jax
import jax.numpy as jnp
from jax.experimental import pallas as pl


def kernel(x, edge_index, emb_a, emb_b, W1, b1, W2, b2):
    raise NotImplementedError("write your pallas kernel here")


# speedup vs baseline: 38.3111x; 38.3111x over previous
import functools

import jax
import jax.numpy as jnp
from jax import lax
from jax.experimental import pallas as pl
from jax.experimental.pallas import tpu as pltpu
from jax.experimental.pallas import tpu_sc as plsc

N = 50000
E = 1600000
OUT = 16
HID = 32
LANE = 128

N_ROWS = 400
N_PAD = N_ROWS * LANE
E_ROWS = 12544
E_PAD = E_ROWS * LANE
NC, NS = 2, 16
NW = NC * NS
TILE_E_ROWS = E_ROWS // NW
NPT = N_PAD // NS
CH = 56
G = 4

_mesh = plsc.VectorSubcoreMesh(core_axis_name="c", subcore_axis_name="s")
_f32 = jnp.float32
_sc_params = pltpu.CompilerParams(use_tc_tiling_on_sc=False)


def _sc_deg_emb_body(dst_hbm, ia_hbm, ib_hbm, ea_hbm, eb_hbm, zd_hbm,
                     deg_hbm, ca_hbm, cb_hbm,
                     idx_v, nidx_v, erows_v, ones_v, gsem, deg_sp):
    c = lax.axis_index("c")
    s = lax.axis_index("s")

    for i in range(LANE // 16):
        ones_v[pl.ds(i * 16, 16)] = jnp.ones((16,), _f32)

    pltpu.sync_copy(zd_hbm, deg_sp.at[pl.ds(s * NPT, NPT)])
    plsc.subcore_barrier()

    base = (c * NS + s) * TILE_E_ROWS

    @pl.loop(0, TILE_E_ROWS // CH)
    def _deg_chunk(k):
        pltpu.sync_copy(dst_hbm.at[pl.ds(base + k * CH, CH)], idx_v)

        @pl.loop(0, CH)
        def _deg_row(j):
            pltpu.sync_copy(ones_v, deg_sp.at[idx_v.at[j]], add=True)

    plsc.subcore_barrier()
    pltpu.sync_copy(deg_sp.at[pl.ds(s * NPT, NPT)],
                    deg_hbm.at[c, pl.ds(s * NPT, NPT)])

    w = s * NC + c
    nrt = 16
    row0 = jnp.minimum(w * nrt, N_ROWS - nrt)
    pltpu.sync_copy(ia_hbm.at[pl.ds(row0, nrt)], nidx_v)

    @pl.loop(0, nrt)
    def _ga(j):
        pltpu.async_copy(ea_hbm.at[nidx_v.at[j]], erows_v, gsem).wait()
        pltpu.sync_copy(erows_v, ca_hbm.at[pl.ds((row0 + j) * LANE, LANE)])

    pltpu.sync_copy(ib_hbm.at[pl.ds(row0, nrt)], nidx_v)

    @pl.loop(0, nrt)
    def _gb(j):
        pltpu.async_copy(eb_hbm.at[nidx_v.at[j]], erows_v, gsem).wait()
        pltpu.sync_copy(erows_v, cb_hbm.at[pl.ds((row0 + j) * LANE, LANE)])


def _sc_deg_emb(dst_r, ia_r, ib_r, emb_a, emb_b):
    zd = jnp.zeros((NPT,), _f32)
    return pl.kernel(
        _sc_deg_emb_body,
        out_type=[
            jax.ShapeDtypeStruct((NC, N_PAD), _f32),
            jax.ShapeDtypeStruct((N_PAD, OUT), _f32),
            jax.ShapeDtypeStruct((N_PAD, OUT), _f32),
        ],
        mesh=_mesh,
        compiler_params=_sc_params,
        scratch_types=[
            pltpu.VMEM((CH, LANE), jnp.int32),
            pltpu.VMEM((16, LANE), jnp.int32),
            pltpu.VMEM((LANE, OUT), _f32),
            pltpu.VMEM((LANE,), _f32),
            pltpu.SemaphoreType.DMA,
            pltpu.VMEM_SHARED((N_PAD,), _f32),
        ],
    )(dst_r, ia_r, ib_r, emb_a, emb_b, zd)


def _sc_scatter_body(C, src_hbm, dst_hbm, g_hbm, z_hbm, out_hbm,
                     sidx_v, didx_v, rows_v, gsem, acc_sp):
    if True:
        c = lax.axis_index("c")
        s = lax.axis_index("s")

        pltpu.sync_copy(z_hbm, acc_sp.at[pl.ds(s * NPT, NPT)])
        plsc.subcore_barrier()

        base = (c * NS + s) * TILE_E_ROWS

        @pl.loop(0, TILE_E_ROWS // CH)
        def _chunk(k):
            pltpu.sync_copy(src_hbm.at[pl.ds(base + k * CH, CH)], sidx_v)
            pltpu.sync_copy(dst_hbm.at[pl.ds(base + k * CH, CH)], didx_v)

            @pl.loop(0, CH // G)
            def _group(gi):
                for b in range(G):
                    pltpu.async_copy(g_hbm.at[sidx_v.at[gi * G + b]],
                                     rows_v.at[b], gsem)
                for b in range(G):
                    pltpu.make_async_copy(g_hbm.at[sidx_v.at[gi * G + b]],
                                          rows_v.at[b], gsem).wait()
                    pltpu.sync_copy(rows_v.at[b],
                                    acc_sp.at[didx_v.at[gi * G + b]], add=True)

        plsc.subcore_barrier()
        pltpu.sync_copy(acc_sp.at[pl.ds(s * NPT, NPT)],
                        out_hbm.at[c, pl.ds(s * NPT, NPT)])


def _sc_scatter(src_r, dst_r, g):
    C = OUT
    z = jnp.zeros((NPT, C), _f32)
    return pl.kernel(
        functools.partial(_sc_scatter_body, C),
        out_type=jax.ShapeDtypeStruct((NC, N_PAD, C), _f32),
        mesh=_mesh,
        compiler_params=_sc_params,
        scratch_types=[
            pltpu.VMEM((CH, LANE), jnp.int32),
            pltpu.VMEM((CH, LANE), jnp.int32),
            pltpu.VMEM((G, LANE, C), _f32),
            pltpu.SemaphoreType.DMA,
            pltpu.VMEM_SHARED((N_PAD, C), _f32),
        ],
    )(src_r, dst_r, g, z)


BN = 6400


def _tc_g1_body(degp, ca, cb, xp, W1, dis_o, g1a_o, g1b_o):
    deg = degp[0] + degp[1] + 1.0
    dis = lax.rsqrt(deg)
    h1 = (jnp.dot(ca[...], W1[0:16, :], preferred_element_type=_f32)
          + jnp.dot(cb[...], W1[16:32, :], preferred_element_type=_f32)
          + jnp.dot(xp[...][:, 2:6], W1[32:36, :], preferred_element_type=_f32))
    g1 = dis * h1
    dis_o[...] = dis
    g1a_o[...] = g1[:, 0:OUT]
    g1b_o[...] = g1[:, OUT:HID]


def _tc_g1(degp, ca, cb, xp, W1):
    grid = N_PAD // BN
    return pl.pallas_call(
        _tc_g1_body,
        grid=(grid,),
        in_specs=[
            pl.BlockSpec((NC, BN, 1), lambda i: (0, i, 0)),
            pl.BlockSpec((BN, OUT), lambda i: (i, 0)),
            pl.BlockSpec((BN, OUT), lambda i: (i, 0)),
            pl.BlockSpec((BN, 6), lambda i: (i, 0)),
            pl.BlockSpec((36, HID), lambda i: (0, 0)),
        ],
        out_specs=[
            pl.BlockSpec((BN, 1), lambda i: (i, 0)),
            pl.BlockSpec((BN, OUT), lambda i: (i, 0)),
            pl.BlockSpec((BN, OUT), lambda i: (i, 0)),
        ],
        out_shape=[
            jax.ShapeDtypeStruct((N_PAD, 1), _f32),
            jax.ShapeDtypeStruct((N_PAD, OUT), _f32),
            jax.ShapeDtypeStruct((N_PAD, OUT), _f32),
        ],
    )(degp, ca, cb, xp, W1)


def _tc_g2_body(s1a, s1b, g1a, g1b, dis, b1, W2, g2_o):
    ta = s1a[0] + s1a[1] + g1a[...]
    tb = s1b[0] + s1b[1] + g1b[...]
    t = jnp.concatenate([ta, tb], axis=-1)
    out1 = jnp.maximum(dis[...] * t + b1[...], 0.0)
    g2_o[...] = dis[...] * jnp.dot(out1, W2[...], preferred_element_type=_f32)


def _tc_g2(s1a, s1b, g1a, g1b, dis, b1, W2):
    grid = N_PAD // BN
    return pl.pallas_call(
        _tc_g2_body,
        grid=(grid,),
        in_specs=[
            pl.BlockSpec((NC, BN, OUT), lambda i: (0, i, 0)),
            pl.BlockSpec((NC, BN, OUT), lambda i: (0, i, 0)),
            pl.BlockSpec((BN, OUT), lambda i: (i, 0)),
            pl.BlockSpec((BN, OUT), lambda i: (i, 0)),
            pl.BlockSpec((BN, 1), lambda i: (i, 0)),
            pl.BlockSpec((1, HID), lambda i: (0, 0)),
            pl.BlockSpec((HID, OUT), lambda i: (0, 0)),
        ],
        out_specs=pl.BlockSpec((BN, OUT), lambda i: (i, 0)),
        out_shape=jax.ShapeDtypeStruct((N_PAD, OUT), _f32),
    )(s1a, s1b, g1a, g1b, dis, b1, W2)


def _tc_out_body(s2, g2, dis, b2, out_o):
    t = s2[0] + s2[1] + g2[...]
    out_o[...] = jnp.maximum(dis[...] * t + b2[...], 0.0)


def _tc_out(s2, g2, dis, b2):
    grid = N_PAD // BN
    return pl.pallas_call(
        _tc_out_body,
        grid=(grid,),
        in_specs=[
            pl.BlockSpec((NC, BN, OUT), lambda i: (0, i, 0)),
            pl.BlockSpec((BN, OUT), lambda i: (i, 0)),
            pl.BlockSpec((BN, 1), lambda i: (i, 0)),
            pl.BlockSpec((1, OUT), lambda i: (0, 0)),
        ],
        out_specs=pl.BlockSpec((BN, OUT), lambda i: (i, 0)),
        out_shape=jax.ShapeDtypeStruct((N_PAD, OUT), _f32),
    )(s2, g2, dis, b2)


def kernel(x, edge_index, emb_a, emb_b, W1, b1, W2, b2):
    src = edge_index[0].astype(jnp.int32)
    dst = edge_index[1].astype(jnp.int32)
    pad_e = E_PAD - E
    pad_src = jnp.zeros((pad_e,), jnp.int32)
    pad_dst = N + (jnp.arange(pad_e, dtype=jnp.int32) % (N_PAD - N))
    src_r = jnp.concatenate([src, pad_src]).reshape(E_ROWS, LANE)
    dst_r = jnp.concatenate([dst, pad_dst]).reshape(E_ROWS, LANE)

    pad_n = N_PAD - N
    ia = x[:, 0].astype(jnp.int32)
    ib = x[:, 1].astype(jnp.int32)
    zi = jnp.zeros((pad_n,), jnp.int32)
    ia_r = jnp.concatenate([ia, zi]).reshape(N_ROWS, LANE)
    ib_r = jnp.concatenate([ib, zi]).reshape(N_ROWS, LANE)
    xp = jnp.pad(x, ((0, pad_n), (0, 0)))

    degp, ca, cb = _sc_deg_emb(dst_r, ia_r, ib_r, emb_a, emb_b)
    dis, g1a, g1b = _tc_g1(degp.reshape(NC, N_PAD, 1), ca, cb, xp, W1)
    s1a = _sc_scatter(src_r, dst_r, g1a)
    s1b = _sc_scatter(src_r, dst_r, g1b)
    g2 = _tc_g2(s1a, s1b, g1a, g1b, dis, b1.reshape(1, HID), W2)
    s2 = _sc_scatter(src_r, dst_r, g2)
    out = _tc_out(s2, g2, dis, b2.reshape(1, OUT))
    return out[:N]


# speedup vs baseline: 47.3270x; 1.2353x over previous
import functools

import jax
import jax.numpy as jnp
from jax import lax
from jax.experimental import pallas as pl
from jax.experimental.pallas import tpu as pltpu
from jax.experimental.pallas import tpu_sc as plsc

N = 50000
E = 1600000
OUT = 16
HID = 32
LANE = 128

N_ROWS = 400
N_PAD = N_ROWS * LANE
E_ROWS = 12544
NC, NS = 2, 16
NW = NC * NS
TILE_E_ROWS = E_ROWS // NW
TILE_E_ROWS2 = E_ROWS // NS
NPT = N_PAD // NS
CH = 56
G = 4
NGRP = CH // G

_mesh = plsc.VectorSubcoreMesh(core_axis_name="c", subcore_axis_name="s")
_f32 = jnp.float32
_sc_params = pltpu.CompilerParams(use_tc_tiling_on_sc=False)


def _edge_pipeline(g_hbm, src_hbm, dst_hbm, acc_sp,
                   sidx_v, didx_v, rows_v, gsems, ssems, base, nchunks):
    def FG(g, h):
        for b in range(G):
            pltpu.async_copy(g_hbm.at[sidx_v.at[g * G + b]],
                             rows_v.at[h, b], gsems[h])

    def WG(g, h):
        for b in range(G):
            pltpu.make_async_copy(g_hbm.at[sidx_v.at[g * G + b]],
                                  rows_v.at[h, b], gsems[h]).wait()

    def FS(g, h):
        for b in range(G):
            pltpu.async_copy(rows_v.at[h, b],
                             acc_sp.at[didx_v.at[g * G + b]], ssems[h],
                             add=True)

    def WS(g, h):
        for b in range(G):
            pltpu.make_async_copy(rows_v.at[h, b],
                                  acc_sp.at[didx_v.at[g * G + b]],
                                  ssems[h]).wait()

    @pl.loop(0, nchunks)
    def _chunk(k):
        row0 = base + k * CH
        pltpu.sync_copy(src_hbm.at[pl.ds(row0, CH)], sidx_v)
        pltpu.sync_copy(dst_hbm.at[pl.ds(row0, CH)], didx_v)
        FG(0, 0)
        FG(1, 1)

        @pl.loop(0, NGRP // 2 - 1)
        def _pair(p):
            g0 = 2 * p
            WG(g0, 0)
            FS(g0, 0)
            WG(g0 + 1, 1)
            FS(g0 + 1, 1)
            WS(g0, 0)
            FG(g0 + 2, 0)
            WS(g0 + 1, 1)
            FG(g0 + 3, 1)

        WG(NGRP - 2, 0)
        FS(NGRP - 2, 0)
        WG(NGRP - 1, 1)
        FS(NGRP - 1, 1)
        WS(NGRP - 2, 0)
        WS(NGRP - 1, 1)


def _sc_deg_emb_body(dst_hbm, ia_hbm, ib_hbm, ea_hbm, eb_hbm, zd_hbm,
                     deg_hbm, ca_hbm, cb_hbm,
                     idx_v, nidx_v, erows_v, ones_v, sem0, sem1, deg_sp):
    c = lax.axis_index("c")
    s = lax.axis_index("s")

    for i in range(LANE // 16):
        ones_v[pl.ds(i * 16, 16)] = jnp.ones((16,), _f32)

    pltpu.sync_copy(zd_hbm, deg_sp.at[pl.ds(s * NPT, NPT)])
    plsc.subcore_barrier()

    base = (c * NS + s) * TILE_E_ROWS
    LAG = 8

    def fire(j):
        pltpu.async_copy(ones_v, deg_sp.at[idx_v.at[j]], sem0, add=True)

    def drain(j):
        pltpu.make_async_copy(ones_v, deg_sp.at[idx_v.at[j]], sem0).wait()

    @pl.loop(0, TILE_E_ROWS // CH)
    def _deg_chunk(k):
        pltpu.sync_copy(dst_hbm.at[pl.ds(base + k * CH, CH)], idx_v)
        for j in range(LAG):
            fire(j)

        @pl.loop(LAG, CH)
        def _deg_row(j):
            fire(j)
            drain(j - LAG)

        for j in range(CH - LAG, CH):
            drain(j)

    plsc.subcore_barrier()
    pltpu.sync_copy(deg_sp.at[pl.ds(s * NPT, NPT)],
                    deg_hbm.at[c, pl.ds(s * NPT, NPT)])

    w = s * NC + c
    nrt = 16
    row0 = jnp.minimum(w * nrt, N_ROWS - nrt)

    def _emb_gather(tbl_hbm, nidx, out_hbm):
        sems = (sem0, sem1)

        def fg(j, h):
            pltpu.async_copy(tbl_hbm.at[nidx.at[j]], erows_v.at[h], sems[h])

        def wg_wr(j, h):
            pltpu.make_async_copy(tbl_hbm.at[nidx.at[j]],
                                  erows_v.at[h], sems[h]).wait()
            pltpu.sync_copy(erows_v.at[h],
                            out_hbm.at[pl.ds((row0 + j) * LANE, LANE)])

        fg(0, 0)
        fg(1, 1)
        for p in range(nrt // 2 - 1):
            wg_wr(2 * p, 0)
            fg(2 * p + 2, 0)
            wg_wr(2 * p + 1, 1)
            fg(2 * p + 3, 1)
        wg_wr(nrt - 2, 0)
        wg_wr(nrt - 1, 1)

    pltpu.sync_copy(ia_hbm.at[pl.ds(row0, nrt)], nidx_v)
    _emb_gather(ea_hbm, nidx_v, ca_hbm)
    pltpu.sync_copy(ib_hbm.at[pl.ds(row0, nrt)], nidx_v)
    _emb_gather(eb_hbm, nidx_v, cb_hbm)


def _sc_deg_emb(dst_r, ia_r, ib_r, emb_a, emb_b):
    zd = jnp.zeros((NPT,), _f32)
    return pl.kernel(
        _sc_deg_emb_body,
        out_type=[
            jax.ShapeDtypeStruct((NC, N_PAD), _f32),
            jax.ShapeDtypeStruct((N_PAD, OUT), _f32),
            jax.ShapeDtypeStruct((N_PAD, OUT), _f32),
        ],
        mesh=_mesh,
        compiler_params=_sc_params,
        scratch_types=[
            pltpu.VMEM((CH, LANE), jnp.int32),
            pltpu.VMEM((16, LANE), jnp.int32),
            pltpu.VMEM((2, LANE, OUT), _f32),
            pltpu.VMEM((LANE,), _f32),
            pltpu.SemaphoreType.DMA,
            pltpu.SemaphoreType.DMA,
            pltpu.VMEM_SHARED((N_PAD,), _f32),
        ],
    )(dst_r, ia_r, ib_r, emb_a, emb_b, zd)


def _sc_scatter1_body(src_hbm, dst_hbm, ga_hbm, gb_hbm, z_hbm, out_hbm,
                      sidx_v, didx_v, rows_v, gsem0, gsem1, ssem0, ssem1,
                      acc_sp):
    c = lax.axis_index("c")
    s = lax.axis_index("s")

    pltpu.sync_copy(z_hbm, acc_sp.at[pl.ds(s * NPT, NPT)])
    plsc.subcore_barrier()

    args = (src_hbm, dst_hbm, acc_sp, sidx_v, didx_v, rows_v,
            (gsem0, gsem1), (ssem0, ssem1))

    @pl.when(c == 0)
    def _half_a():
        _edge_pipeline(ga_hbm, *args[0:2], *args[2:],
                       base=s * TILE_E_ROWS2, nchunks=TILE_E_ROWS2 // CH)

    @pl.when(c == 1)
    def _half_b():
        _edge_pipeline(gb_hbm, *args[0:2], *args[2:],
                       base=s * TILE_E_ROWS2, nchunks=TILE_E_ROWS2 // CH)

    plsc.subcore_barrier()
    pltpu.sync_copy(acc_sp.at[pl.ds(s * NPT, NPT)],
                    out_hbm.at[c, pl.ds(s * NPT, NPT)])


def _sc_scatter1(src_r, dst_r, ga, gb):
    z = jnp.zeros((NPT, OUT), _f32)
    return pl.kernel(
        _sc_scatter1_body,
        out_type=jax.ShapeDtypeStruct((NC, N_PAD, OUT), _f32),
        mesh=_mesh,
        compiler_params=_sc_params,
        scratch_types=[
            pltpu.VMEM((CH, LANE), jnp.int32),
            pltpu.VMEM((CH, LANE), jnp.int32),
            pltpu.VMEM((2, G, LANE, OUT), _f32),
            pltpu.SemaphoreType.DMA,
            pltpu.SemaphoreType.DMA,
            pltpu.SemaphoreType.DMA,
            pltpu.SemaphoreType.DMA,
            pltpu.VMEM_SHARED((N_PAD, OUT), _f32),
        ],
    )(src_r, dst_r, ga, gb, z)


def _sc_scatter2_body(src_hbm, dst_hbm, g_hbm, z_hbm, out_hbm,
                      sidx_v, didx_v, rows_v, gsem0, gsem1, ssem0, ssem1,
                      acc_sp):
    c = lax.axis_index("c")
    s = lax.axis_index("s")

    pltpu.sync_copy(z_hbm, acc_sp.at[pl.ds(s * NPT, NPT)])
    plsc.subcore_barrier()

    _edge_pipeline(g_hbm, src_hbm, dst_hbm, acc_sp, sidx_v, didx_v, rows_v,
                   (gsem0, gsem1), (ssem0, ssem1),
                   base=(c * NS + s) * TILE_E_ROWS,
                   nchunks=TILE_E_ROWS // CH)

    plsc.subcore_barrier()
    pltpu.sync_copy(acc_sp.at[pl.ds(s * NPT, NPT)],
                    out_hbm.at[c, pl.ds(s * NPT, NPT)])


def _sc_scatter2(src_r, dst_r, g):
    z = jnp.zeros((NPT, OUT), _f32)
    return pl.kernel(
        _sc_scatter2_body,
        out_type=jax.ShapeDtypeStruct((NC, N_PAD, OUT), _f32),
        mesh=_mesh,
        compiler_params=_sc_params,
        scratch_types=[
            pltpu.VMEM((CH, LANE), jnp.int32),
            pltpu.VMEM((CH, LANE), jnp.int32),
            pltpu.VMEM((2, G, LANE, OUT), _f32),
            pltpu.SemaphoreType.DMA,
            pltpu.SemaphoreType.DMA,
            pltpu.SemaphoreType.DMA,
            pltpu.SemaphoreType.DMA,
            pltpu.VMEM_SHARED((N_PAD, OUT), _f32),
        ],
    )(src_r, dst_r, g, z)


BN = 6400


def _tc_g1_body(degp, ca, cb, xp, W1, dis_o, g1a_o, g1b_o):
    deg = degp[0] + degp[1] + 1.0
    dis = lax.rsqrt(deg)
    h1 = (jnp.dot(ca[...], W1[0:16, :], preferred_element_type=_f32)
          + jnp.dot(cb[...], W1[16:32, :], preferred_element_type=_f32)
          + jnp.dot(xp[...][:, 2:6], W1[32:36, :], preferred_element_type=_f32))
    g1 = dis * h1
    dis_o[...] = dis
    g1a_o[...] = g1[:, 0:OUT]
    g1b_o[...] = g1[:, OUT:HID]


def _tc_g1(degp, ca, cb, xp, W1):
    grid = N_PAD // BN
    return pl.pallas_call(
        _tc_g1_body,
        grid=(grid,),
        in_specs=[
            pl.BlockSpec((NC, BN, 1), lambda i: (0, i, 0)),
            pl.BlockSpec((BN, OUT), lambda i: (i, 0)),
            pl.BlockSpec((BN, OUT), lambda i: (i, 0)),
            pl.BlockSpec((BN, 6), lambda i: (i, 0)),
            pl.BlockSpec((36, HID), lambda i: (0, 0)),
        ],
        out_specs=[
            pl.BlockSpec((BN, 1), lambda i: (i, 0)),
            pl.BlockSpec((BN, OUT), lambda i: (i, 0)),
            pl.BlockSpec((BN, OUT), lambda i: (i, 0)),
        ],
        out_shape=[
            jax.ShapeDtypeStruct((N_PAD, 1), _f32),
            jax.ShapeDtypeStruct((N_PAD, OUT), _f32),
            jax.ShapeDtypeStruct((N_PAD, OUT), _f32),
        ],
    )(degp, ca, cb, xp, W1)


def _tc_g2_body(s1, g1a, g1b, dis, b1, W2, g2_o):
    ta = s1[0] + g1a[...]
    tb = s1[1] + g1b[...]
    t = jnp.concatenate([ta, tb], axis=-1)
    out1 = jnp.maximum(dis[...] * t + b1[...], 0.0)
    g2_o[...] = dis[...] * jnp.dot(out1, W2[...], preferred_element_type=_f32)


def _tc_g2(s1, g1a, g1b, dis, b1, W2):
    grid = N_PAD // BN
    return pl.pallas_call(
        _tc_g2_body,
        grid=(grid,),
        in_specs=[
            pl.BlockSpec((NC, BN, OUT), lambda i: (0, i, 0)),
            pl.BlockSpec((BN, OUT), lambda i: (i, 0)),
            pl.BlockSpec((BN, OUT), lambda i: (i, 0)),
            pl.BlockSpec((BN, 1), lambda i: (i, 0)),
            pl.BlockSpec((1, HID), lambda i: (0, 0)),
            pl.BlockSpec((HID, OUT), lambda i: (0, 0)),
        ],
        out_specs=pl.BlockSpec((BN, OUT), lambda i: (i, 0)),
        out_shape=jax.ShapeDtypeStruct((N_PAD, OUT), _f32),
    )(s1, g1a, g1b, dis, b1, W2)


def _tc_out_body(s2, g2, dis, b2, out_o):
    t = s2[0] + s2[1] + g2[...]
    out_o[...] = jnp.maximum(dis[...] * t + b2[...], 0.0)


def _tc_out(s2, g2, dis, b2):
    grid = N_PAD // BN
    return pl.pallas_call(
        _tc_out_body,
        grid=(grid,),
        in_specs=[
            pl.BlockSpec((NC, BN, OUT), lambda i: (0, i, 0)),
            pl.BlockSpec((BN, OUT), lambda i: (i, 0)),
            pl.BlockSpec((BN, 1), lambda i: (i, 0)),
            pl.BlockSpec((1, OUT), lambda i: (0, 0)),
        ],
        out_specs=pl.BlockSpec((BN, OUT), lambda i: (i, 0)),
        out_shape=jax.ShapeDtypeStruct((N_PAD, OUT), _f32),
    )(s2, g2, dis, b2)


def kernel(x, edge_index, emb_a, emb_b, W1, b1, W2, b2):
    src = edge_index[0].astype(jnp.int32)
    dst = edge_index[1].astype(jnp.int32)
    pad_e = E_ROWS * LANE - E
    pad_src = jnp.zeros((pad_e,), jnp.int32)
    pad_dst = N + (jnp.arange(pad_e, dtype=jnp.int32) % (N_PAD - N))
    src_r = jnp.concatenate([src, pad_src]).reshape(E_ROWS, LANE)
    dst_r = jnp.concatenate([dst, pad_dst]).reshape(E_ROWS, LANE)

    pad_n = N_PAD - N
    ia = x[:, 0].astype(jnp.int32)
    ib = x[:, 1].astype(jnp.int32)
    zi = jnp.zeros((pad_n,), jnp.int32)
    ia_r = jnp.concatenate([ia, zi]).reshape(N_ROWS, LANE)
    ib_r = jnp.concatenate([ib, zi]).reshape(N_ROWS, LANE)
    xp = jnp.pad(x, ((0, pad_n), (0, 0)))

    degp, ca, cb = _sc_deg_emb(dst_r, ia_r, ib_r, emb_a, emb_b)
    dis, g1a, g1b = _tc_g1(degp.reshape(NC, N_PAD, 1), ca, cb, xp, W1)
    s1 = _sc_scatter1(src_r, dst_r, g1a, g1b)
    g2 = _tc_g2(s1, g1a, g1b, dis, b1.reshape(1, HID), W2)
    s2 = _sc_scatter2(src_r, dst_r, g2)
    out = _tc_out(s2, g2, dis, b2.reshape(1, OUT))
    return out[:N]


# speedup vs baseline: 51.0350x; 1.0784x over previous
import functools

import jax
import jax.numpy as jnp
from jax import lax
from jax.experimental import pallas as pl
from jax.experimental.pallas import tpu as pltpu
from jax.experimental.pallas import tpu_sc as plsc

N = 50000
E = 1600000
OUT = 16
HID = 32
LANE = 128

NV = 391
N_ROWS = 400
N_PAD = N_ROWS * LANE
E_ROWS = 12544
NC, NS = 2, 16
NW = NC * NS
TILE_E_ROWS = E_ROWS // NW
TILE_E_ROWS2 = E_ROWS // NS
NPT = N_PAD // NS
CH = 56
G = 4
NGRP = CH // G

_mesh = plsc.VectorSubcoreMesh(core_axis_name="c", subcore_axis_name="s")
_f32 = jnp.float32
_i32 = jnp.int32
_sc_params = pltpu.CompilerParams(use_tc_tiling_on_sc=False,
                                  needs_layout_passes=False)


def _rsqrt16(d):
    xi = plsc.bitcast(d, _i32)
    y = plsc.bitcast(jnp.int32(0x5F3759DF) - (xi >> 1), _f32)
    for _ in range(3):
        y = y * (1.5 - 0.5 * d * y * y)
    return y


def _tc_tables_body(ea, eb, W1, ta_o, tb_o):
    ta_o[...] = jnp.dot(ea[...], W1[0:16, :], preferred_element_type=_f32)
    tb_o[...] = jnp.dot(eb[...], W1[16:32, :], preferred_element_type=_f32)


def _tc_tables(emb_a, emb_b, W1):
    return pl.pallas_call(
        _tc_tables_body,
        out_shape=[
            jax.ShapeDtypeStruct((1000, HID), _f32),
            jax.ShapeDtypeStruct((1000, HID), _f32),
        ],
    )(emb_a, emb_b, W1)


def _sc_deg_body(dst_hbm, zd_hbm, deg_hbm, idx_v, ones_v, sem0, deg_sp):
    c = lax.axis_index("c")
    s = lax.axis_index("s")

    for i in range(LANE // 16):
        ones_v[pl.ds(i * 16, 16)] = jnp.ones((16,), _f32)

    pltpu.sync_copy(zd_hbm, deg_sp.at[pl.ds(s * NPT, NPT)])
    plsc.subcore_barrier()

    base = (c * NS + s) * TILE_E_ROWS
    LAG = 8

    def fire(j):
        pltpu.async_copy(ones_v, deg_sp.at[idx_v.at[j]], sem0, add=True)

    def drain(j):
        pltpu.make_async_copy(ones_v, deg_sp.at[idx_v.at[j]], sem0).wait()

    @pl.loop(0, TILE_E_ROWS // CH)
    def _deg_chunk(k):
        pltpu.sync_copy(dst_hbm.at[pl.ds(base + k * CH, CH)], idx_v)
        for j in range(LAG):
            fire(j)

        @pl.loop(LAG, CH)
        def _deg_row(j):
            fire(j)
            drain(j - LAG)

        for j in range(CH - LAG, CH):
            drain(j)

    plsc.subcore_barrier()
    pltpu.sync_copy(deg_sp.at[pl.ds(s * NPT, NPT)],
                    deg_hbm.at[c, pl.ds(s * NPT, NPT)])


def _sc_deg(dst_r):
    zd = jnp.zeros((NPT,), _f32)
    return pl.kernel(
        _sc_deg_body,
        out_type=jax.ShapeDtypeStruct((NC, N_PAD), _f32),
        mesh=_mesh,
        compiler_params=_sc_params,
        scratch_types=[
            pltpu.VMEM((CH, LANE), _i32),
            pltpu.VMEM((LANE,), _f32),
            pltpu.SemaphoreType.DMA,
            pltpu.VMEM_SHARED((N_PAD,), _f32),
        ],
    )(dst_r, zd)


def _sc_node1_body(x_hbm, ta_hbm, tb_hbm, W1_hbm, degp_hbm,
                   dis_hbm, g1a_hbm, g1b_hbm,
                   ta_v, tb_v, x_v, w1_v, deg_v, dis_v, ga_v, gb_v):
    c = lax.axis_index("c")
    s = lax.axis_index("s")
    w = s * NC + c

    pltpu.sync_copy(ta_hbm, ta_v)
    pltpu.sync_copy(tb_hbm, tb_v)
    pltpu.sync_copy(W1_hbm, w1_v)
    w1a = [w1_v[32 + k, pl.ds(0, 16)] for k in range(4)]
    w1b = [w1_v[32 + k, pl.ds(16, 16)] for k in range(4)]
    iota = lax.iota(_i32, 16)

    @pl.loop(w * NV // NW, (w + 1) * NV // NW)
    def _row(r):
        node0 = jnp.minimum(r * LANE, N - LANE)
        pltpu.sync_copy(x_hbm.at[pl.ds(node0, LANE)], x_v)
        pltpu.sync_copy(degp_hbm.at[0, pl.ds(node0, LANE)], deg_v.at[0])
        pltpu.sync_copy(degp_hbm.at[1, pl.ds(node0, LANE)], deg_v.at[1])
        for v in range(LANE // 16):
            sl = pl.ds(v * 16, 16)
            d = deg_v[0, sl] + deg_v[1, sl] + 1.0
            dis_v[sl] = _rsqrt16(d)
        pltpu.sync_copy(dis_v, dis_hbm.at[pl.ds(node0, LANE)])

        @pl.loop(0, LANE // 16)
        def _grp(t):
            nidx = iota + t * 16
            dis16 = dis_v[pl.ds(t * 16, 16)]
            ia16 = plsc.load_gather(
                x_v, [nidx, jnp.zeros((16,), _i32)]).astype(_i32)
            ib16 = plsc.load_gather(
                x_v, [nidx, jnp.full((16,), 1, _i32)]).astype(_i32)
            nums = [plsc.load_gather(x_v, [nidx, jnp.full((16,), 2 + k, _i32)])
                    for k in range(4)]
            for col in range(HID):
                cc = jnp.full((16,), col, _i32)
                h = (plsc.load_gather(ta_v, [ia16, cc])
                     + plsc.load_gather(tb_v, [ib16, cc]))
                for k in range(4):
                    wk = w1a[k][col] if col < 16 else w1b[k][col - 16]
                    h = h + nums[k] * wk
                g = h * dis16
                if col < 16:
                    plsc.store_scatter(ga_v, [nidx, cc], g)
                else:
                    plsc.store_scatter(
                        gb_v, [nidx, jnp.full((16,), col - 16, _i32)], g)

        pltpu.sync_copy(ga_v, g1a_hbm.at[pl.ds(node0, LANE)])
        pltpu.sync_copy(gb_v, g1b_hbm.at[pl.ds(node0, LANE)])


def _sc_node1(x, ta, tb, W1, degp):
    return pl.kernel(
        _sc_node1_body,
        out_type=[
            jax.ShapeDtypeStruct((N_PAD,), _f32),
            jax.ShapeDtypeStruct((N_PAD, OUT), _f32),
            jax.ShapeDtypeStruct((N_PAD, OUT), _f32),
        ],
        mesh=_mesh,
        compiler_params=_sc_params,
        scratch_types=[
            pltpu.VMEM((1000, HID), _f32),
            pltpu.VMEM((1000, HID), _f32),
            pltpu.VMEM((LANE, 6), _f32),
            pltpu.VMEM((36, HID), _f32),
            pltpu.VMEM((2, LANE), _f32),
            pltpu.VMEM((LANE,), _f32),
            pltpu.VMEM((LANE, OUT), _f32),
            pltpu.VMEM((LANE, OUT), _f32),
        ],
    )(x, ta, tb, W1, degp)


def _edge_pipeline(g_hbm, src_hbm, dst_hbm, acc_sp,
                   sidx_v, didx_v, rows_v, gsems, ssems, base, nchunks):
    def FG(g, h):
        for b in range(G):
            pltpu.async_copy(g_hbm.at[sidx_v.at[g * G + b]],
                             rows_v.at[h, b], gsems[h])

    def WG(g, h):
        for b in range(G):
            pltpu.make_async_copy(g_hbm.at[sidx_v.at[g * G + b]],
                                  rows_v.at[h, b], gsems[h]).wait()

    def FS(g, h):
        for b in range(G):
            pltpu.async_copy(rows_v.at[h, b],
                             acc_sp.at[didx_v.at[g * G + b]], ssems[h],
                             add=True)

    def WS(g, h):
        for b in range(G):
            pltpu.make_async_copy(rows_v.at[h, b],
                                  acc_sp.at[didx_v.at[g * G + b]],
                                  ssems[h]).wait()

    @pl.loop(0, nchunks)
    def _chunk(k):
        row0 = base + k * CH
        pltpu.sync_copy(src_hbm.at[pl.ds(row0, CH)], sidx_v)
        pltpu.sync_copy(dst_hbm.at[pl.ds(row0, CH)], didx_v)
        FG(0, 0)
        FG(1, 1)

        @pl.loop(0, NGRP // 2 - 1)
        def _pair(p):
            g0 = 2 * p
            WG(g0, 0)
            FS(g0, 0)
            WG(g0 + 1, 1)
            FS(g0 + 1, 1)
            WS(g0, 0)
            FG(g0 + 2, 0)
            WS(g0 + 1, 1)
            FG(g0 + 3, 1)

        WG(NGRP - 2, 0)
        FS(NGRP - 2, 0)
        WG(NGRP - 1, 1)
        FS(NGRP - 1, 1)
        WS(NGRP - 2, 0)
        WS(NGRP - 1, 1)


def _sc_scatter1_body(src_hbm, dst_hbm, ga_hbm, gb_hbm, z_hbm, out_hbm,
                      sidx_v, didx_v, rows_v, gsem0, gsem1, ssem0, ssem1,
                      acc_sp):
    c = lax.axis_index("c")
    s = lax.axis_index("s")

    pltpu.sync_copy(z_hbm, acc_sp.at[pl.ds(s * NPT, NPT)])
    plsc.subcore_barrier()

    args = (src_hbm, dst_hbm, acc_sp, sidx_v, didx_v, rows_v,
            (gsem0, gsem1), (ssem0, ssem1))

    @pl.when(c == 0)
    def _half_a():
        _edge_pipeline(ga_hbm, *args,
                       base=s * TILE_E_ROWS2, nchunks=TILE_E_ROWS2 // CH)

    @pl.when(c == 1)
    def _half_b():
        _edge_pipeline(gb_hbm, *args,
                       base=s * TILE_E_ROWS2, nchunks=TILE_E_ROWS2 // CH)

    plsc.subcore_barrier()
    pltpu.sync_copy(acc_sp.at[pl.ds(s * NPT, NPT)],
                    out_hbm.at[c, pl.ds(s * NPT, NPT)])


def _sc_scatter1(src_r, dst_r, ga, gb):
    z = jnp.zeros((NPT, OUT), _f32)
    return pl.kernel(
        _sc_scatter1_body,
        out_type=jax.ShapeDtypeStruct((NC, N_PAD, OUT), _f32),
        mesh=_mesh,
        compiler_params=_sc_params,
        scratch_types=[
            pltpu.VMEM((CH, LANE), _i32),
            pltpu.VMEM((CH, LANE), _i32),
            pltpu.VMEM((2, G, LANE, OUT), _f32),
            pltpu.SemaphoreType.DMA,
            pltpu.SemaphoreType.DMA,
            pltpu.SemaphoreType.DMA,
            pltpu.SemaphoreType.DMA,
            pltpu.VMEM_SHARED((N_PAD, OUT), _f32),
        ],
    )(src_r, dst_r, ga, gb, z)


def _sc_scatter2_body(src_hbm, dst_hbm, g_hbm, z_hbm, out_hbm,
                      sidx_v, didx_v, rows_v, gsem0, gsem1, ssem0, ssem1,
                      acc_sp):
    c = lax.axis_index("c")
    s = lax.axis_index("s")

    pltpu.sync_copy(z_hbm, acc_sp.at[pl.ds(s * NPT, NPT)])
    plsc.subcore_barrier()

    _edge_pipeline(g_hbm, src_hbm, dst_hbm, acc_sp, sidx_v, didx_v, rows_v,
                   (gsem0, gsem1), (ssem0, ssem1),
                   base=(c * NS + s) * TILE_E_ROWS,
                   nchunks=TILE_E_ROWS // CH)

    plsc.subcore_barrier()
    pltpu.sync_copy(acc_sp.at[pl.ds(s * NPT, NPT)],
                    out_hbm.at[c, pl.ds(s * NPT, NPT)])


def _sc_scatter2(src_r, dst_r, g):
    z = jnp.zeros((NPT, OUT), _f32)
    return pl.kernel(
        _sc_scatter2_body,
        out_type=jax.ShapeDtypeStruct((NC, N_PAD, OUT), _f32),
        mesh=_mesh,
        compiler_params=_sc_params,
        scratch_types=[
            pltpu.VMEM((CH, LANE), _i32),
            pltpu.VMEM((CH, LANE), _i32),
            pltpu.VMEM((2, G, LANE, OUT), _f32),
            pltpu.SemaphoreType.DMA,
            pltpu.SemaphoreType.DMA,
            pltpu.SemaphoreType.DMA,
            pltpu.SemaphoreType.DMA,
            pltpu.VMEM_SHARED((N_PAD, OUT), _f32),
        ],
    )(src_r, dst_r, g, z)


def _sc_node2_body(s1_hbm, g1a_hbm, g1b_hbm, dis_hbm, b1_hbm, W2_hbm,
                   g2_hbm,
                   sa_v, sb_v, ga_v, gb_v, dis_v, o1_v, w2_v, b1_v, g2_v):
    c = lax.axis_index("c")
    s = lax.axis_index("s")
    w = s * NC + c

    pltpu.sync_copy(W2_hbm, w2_v)
    pltpu.sync_copy(b1_hbm, b1_v)
    b1a = b1_v[pl.ds(0, 16)]
    b1b = b1_v[pl.ds(16, 16)]
    zero = jnp.zeros((16,), _f32)
    iota = lax.iota(_i32, 16)

    @pl.loop(w * NV // NW, (w + 1) * NV // NW)
    def _row(r):
        node0 = jnp.minimum(r * LANE, N - LANE)
        sl = pl.ds(node0, LANE)
        pltpu.sync_copy(s1_hbm.at[0, sl], sa_v)
        pltpu.sync_copy(s1_hbm.at[1, sl], sb_v)
        pltpu.sync_copy(g1a_hbm.at[sl], ga_v)
        pltpu.sync_copy(g1b_hbm.at[sl], gb_v)
        pltpu.sync_copy(dis_hbm.at[sl], dis_v)

        @pl.loop(0, LANE // 16)
        def _grp(t):
            nidx = iota + t * 16
            dis16 = dis_v[pl.ds(t * 16, 16)]
            for col in range(OUT):
                cc = jnp.full((16,), col, _i32)
                va = (plsc.load_gather(sa_v, [nidx, cc])
                      + plsc.load_gather(ga_v, [nidx, cc]))
                o1_v[col] = jnp.maximum(va * dis16 + b1a[col], zero)
                vb = (plsc.load_gather(sb_v, [nidx, cc])
                      + plsc.load_gather(gb_v, [nidx, cc]))
                o1_v[OUT + col] = jnp.maximum(vb * dis16 + b1b[col], zero)
            accs = [zero] * OUT
            for k in range(HID):
                ok = o1_v[k, :]
                w2k = w2_v[k, :]
                for col in range(OUT):
                    accs[col] = accs[col] + ok * w2k[col]
            for col in range(OUT):
                plsc.store_scatter(g2_v, [nidx, jnp.full((16,), col, _i32)],
                                   accs[col] * dis16)

        pltpu.sync_copy(g2_v, g2_hbm.at[sl])


def _sc_node2(s1, g1a, g1b, dis, b1, W2):
    return pl.kernel(
        _sc_node2_body,
        out_type=jax.ShapeDtypeStruct((N_PAD, OUT), _f32),
        mesh=_mesh,
        compiler_params=_sc_params,
        scratch_types=[
            pltpu.VMEM((LANE, OUT), _f32),
            pltpu.VMEM((LANE, OUT), _f32),
            pltpu.VMEM((LANE, OUT), _f32),
            pltpu.VMEM((LANE, OUT), _f32),
            pltpu.VMEM((LANE,), _f32),
            pltpu.VMEM((HID, 16), _f32),
            pltpu.VMEM((HID, OUT), _f32),
            pltpu.VMEM((HID,), _f32),
            pltpu.VMEM((LANE, OUT), _f32),
        ],
    )(s1, g1a, g1b, dis, b1, W2)


def _sc_node3_body(s2_hbm, g2_hbm, dis_hbm, b2_hbm, out_hbm,
                   sa_v, sb_v, g2_v, dis_v, b2_v, o_v):
    c = lax.axis_index("c")
    s = lax.axis_index("s")
    w = s * NC + c

    pltpu.sync_copy(b2_hbm, b2_v)
    b2r = b2_v[...]
    zero = jnp.zeros((16,), _f32)
    iota = lax.iota(_i32, 16)

    @pl.loop(w * NV // NW, (w + 1) * NV // NW)
    def _row(r):
        node0 = jnp.minimum(r * LANE, N - LANE)
        sl = pl.ds(node0, LANE)
        pltpu.sync_copy(s2_hbm.at[0, sl], sa_v)
        pltpu.sync_copy(s2_hbm.at[1, sl], sb_v)
        pltpu.sync_copy(g2_hbm.at[sl], g2_v)
        pltpu.sync_copy(dis_hbm.at[sl], dis_v)

        @pl.loop(0, LANE // 16)
        def _grp(t):
            nidx = iota + t * 16
            dis16 = dis_v[pl.ds(t * 16, 16)]
            for col in range(OUT):
                cc = jnp.full((16,), col, _i32)
                v = (plsc.load_gather(sa_v, [nidx, cc])
                     + plsc.load_gather(sb_v, [nidx, cc])
                     + plsc.load_gather(g2_v, [nidx, cc]))
                o = jnp.maximum(v * dis16 + b2r[col], zero)
                plsc.store_scatter(o_v, [nidx, cc], o)

        pltpu.sync_copy(o_v, out_hbm.at[sl])


def _sc_node3(s2, g2, dis, b2):
    return pl.kernel(
        _sc_node3_body,
        out_type=jax.ShapeDtypeStruct((N, OUT), _f32),
        mesh=_mesh,
        compiler_params=_sc_params,
        scratch_types=[
            pltpu.VMEM((LANE, OUT), _f32),
            pltpu.VMEM((LANE, OUT), _f32),
            pltpu.VMEM((LANE, OUT), _f32),
            pltpu.VMEM((LANE,), _f32),
            pltpu.VMEM((OUT,), _f32),
            pltpu.VMEM((LANE, OUT), _f32),
        ],
    )(s2, g2, dis, b2)


def kernel(x, edge_index, emb_a, emb_b, W1, b1, W2, b2):
    src = edge_index[0].astype(_i32)
    dst = edge_index[1].astype(_i32)
    pad_e = E_ROWS * LANE - E
    pad_src = jnp.zeros((pad_e,), _i32)
    pad_dst = N + (jnp.arange(pad_e, dtype=_i32) % (N_PAD - N))
    src_r = jnp.concatenate([src, pad_src]).reshape(E_ROWS, LANE)
    dst_r = jnp.concatenate([dst, pad_dst]).reshape(E_ROWS, LANE)

    ta, tb = _tc_tables(emb_a, emb_b, W1)
    degp = _sc_deg(dst_r)
    dis, g1a, g1b = _sc_node1(x, ta, tb, W1, degp)
    s1 = _sc_scatter1(src_r, dst_r, g1a, g1b)
    g2 = _sc_node2(s1, g1a, g1b, dis, b1, W2)
    s2 = _sc_scatter2(src_r, dst_r, g2)
    return _sc_node3(s2, g2, dis, b2)


# speedup vs baseline: 59.3297x; 1.1625x over previous
import functools

import jax
import jax.numpy as jnp
from jax import lax
from jax.experimental import pallas as pl
from jax.experimental.pallas import tpu as pltpu
from jax.experimental.pallas import tpu_sc as plsc

N = 50000
E = 1600000
OUT = 16
HID = 32
LANE = 128

NV = 391
N_ROWS = 400
N_PAD = N_ROWS * LANE
E_ROWS = 12544
NC, NS = 2, 16
NW = NC * NS
TILE_E_ROWS = E_ROWS // NW
TILE_E_ROWS2 = E_ROWS // NS
NPT = N_PAD // NS
CH = 56
G = 4
NGRP = CH // G

_mesh = plsc.VectorSubcoreMesh(core_axis_name="c", subcore_axis_name="s")
_f32 = jnp.float32
_i32 = jnp.int32
_sc_params = pltpu.CompilerParams(use_tc_tiling_on_sc=False,
                                  needs_layout_passes=False)


def _rsqrt16(d):
    xi = plsc.bitcast(d, _i32)
    y = plsc.bitcast(jnp.int32(0x5F3759DF) - (xi >> 1), _f32)
    for _ in range(3):
        y = y * (1.5 - 0.5 * d * y * y)
    return y


def _tc_tables_body(ea, eb, W1, ta_o, tb_o):
    ta_o[...] = jnp.dot(ea[...], W1[0:16, :], preferred_element_type=_f32)
    tb_o[...] = jnp.dot(eb[...], W1[16:32, :], preferred_element_type=_f32)


def _tc_tables(emb_a, emb_b, W1):
    return pl.pallas_call(
        _tc_tables_body,
        out_shape=[
            jax.ShapeDtypeStruct((1000, HID), _f32),
            jax.ShapeDtypeStruct((1000, HID), _f32),
        ],
    )(emb_a, emb_b, W1)


RPT = N_ROWS // 16


def _sc_deg_body(dst_hbm, z2_hbm, deg_hbm, idx_v, ridx_v, ld_v, deg_sp):
    c = lax.axis_index("c")
    s = lax.axis_index("s")
    ones16 = jnp.ones((16,), _f32)
    zeros16 = jnp.zeros((16,), _f32)
    iota = lax.iota(_i32, 16)

    @pl.loop(0, N_ROWS)
    def _z(r):
        for u in range(LANE // 16):
            ld_v[r, pl.ds(u * 16, 16)] = zeros16

    for k in range(RPT):
        ridx_v[k, :] = iota + k * 16

    pltpu.sync_copy(z2_hbm, deg_sp.at[pl.ds(s * RPT, RPT)])
    plsc.subcore_barrier()

    base = (c * NS + s) * TILE_E_ROWS

    @pl.loop(0, TILE_E_ROWS // CH)
    def _deg_chunk(k):
        pltpu.sync_copy(dst_hbm.at[pl.ds(base + k * CH, CH)], idx_v)

        @pl.loop(0, CH)
        def _deg_row(j):
            for u in range(LANE // 16):
                d16 = idx_v[j, pl.ds(u * 16, 16)]
                plsc.addupdate_scatter(ld_v, [d16 >> 7, d16 & 127], ones16)

    @pl.loop(0, RPT)
    def _merge(k):
        pltpu.sync_copy(ld_v.at[pl.ds(k * 16, 16)],
                        deg_sp.at[ridx_v.at[k]], add=True)

    plsc.subcore_barrier()
    pltpu.sync_copy(deg_sp.at[pl.ds(s * RPT, RPT)],
                    deg_hbm.at[c, pl.ds(s * RPT, RPT)])


def _sc_deg(dst_r):
    z2 = jnp.zeros((RPT, LANE), _f32)
    return pl.kernel(
        _sc_deg_body,
        out_type=jax.ShapeDtypeStruct((NC, N_ROWS, LANE), _f32),
        mesh=_mesh,
        compiler_params=_sc_params,
        scratch_types=[
            pltpu.VMEM((CH, LANE), _i32),
            pltpu.VMEM((RPT, 16), _i32),
            pltpu.VMEM((N_ROWS, LANE), _f32),
            pltpu.VMEM_SHARED((N_ROWS, LANE), _f32),
        ],
    )(dst_r, z2)


def _sc_node1_body(x_hbm, ta_hbm, tb_hbm, W1_hbm, degp_hbm,
                   dis_hbm, g1a_hbm, g1b_hbm,
                   ta_v, tb_v, x_v, w1_v, deg_v, dis_v, ga_v, gb_v):
    c = lax.axis_index("c")
    s = lax.axis_index("s")
    w = s * NC + c

    pltpu.sync_copy(ta_hbm, ta_v)
    pltpu.sync_copy(tb_hbm, tb_v)
    pltpu.sync_copy(W1_hbm, w1_v)
    w1a = [w1_v[32 + k, pl.ds(0, 16)] for k in range(4)]
    w1b = [w1_v[32 + k, pl.ds(16, 16)] for k in range(4)]
    iota = lax.iota(_i32, 16)

    @pl.loop(w * NV // NW, (w + 1) * NV // NW)
    def _row(r):
        node0 = jnp.minimum(r * LANE, N - LANE)
        pltpu.sync_copy(x_hbm.at[pl.ds(node0, LANE)], x_v)
        pltpu.sync_copy(degp_hbm.at[0, pl.ds(node0, LANE)], deg_v.at[0])
        pltpu.sync_copy(degp_hbm.at[1, pl.ds(node0, LANE)], deg_v.at[1])
        for v in range(LANE // 16):
            sl = pl.ds(v * 16, 16)
            d = deg_v[0, sl] + deg_v[1, sl] + 1.0
            dis_v[sl] = _rsqrt16(d)
        pltpu.sync_copy(dis_v, dis_hbm.at[pl.ds(node0, LANE)])

        @pl.loop(0, LANE // 16)
        def _grp(t):
            base16 = t * 16
            nidx = iota + base16
            dis16 = dis_v[pl.ds(base16, 16)]
            ia16 = plsc.load_gather(
                x_v, [nidx, jnp.zeros((16,), _i32)]).astype(_i32)
            ib16 = plsc.load_gather(
                x_v, [nidx, jnp.full((16,), 1, _i32)]).astype(_i32)
            nums = [plsc.load_gather(x_v, [nidx, jnp.full((16,), 2 + k, _i32)])
                    for k in range(4)]
            for j in range(16):
                ian = ia16[j]
                ibn = ib16[j]
                ha = ta_v[ian, pl.ds(0, 16)] + tb_v[ibn, pl.ds(0, 16)]
                hb = ta_v[ian, pl.ds(16, 16)] + tb_v[ibn, pl.ds(16, 16)]
                for k in range(4):
                    nk = nums[k][j]
                    ha = ha + nk * w1a[k]
                    hb = hb + nk * w1b[k]
                dn = dis16[j]
                ga_v[base16 + j, :] = ha * dn
                gb_v[base16 + j, :] = hb * dn

        pltpu.sync_copy(ga_v, g1a_hbm.at[pl.ds(node0, LANE)])
        pltpu.sync_copy(gb_v, g1b_hbm.at[pl.ds(node0, LANE)])


def _sc_node1(x, ta, tb, W1, degp):
    return pl.kernel(
        _sc_node1_body,
        out_type=[
            jax.ShapeDtypeStruct((N_PAD,), _f32),
            jax.ShapeDtypeStruct((N_PAD, OUT), _f32),
            jax.ShapeDtypeStruct((N_PAD, OUT), _f32),
        ],
        mesh=_mesh,
        compiler_params=_sc_params,
        scratch_types=[
            pltpu.VMEM((1000, HID), _f32),
            pltpu.VMEM((1000, HID), _f32),
            pltpu.VMEM((LANE, 6), _f32),
            pltpu.VMEM((36, HID), _f32),
            pltpu.VMEM((2, LANE), _f32),
            pltpu.VMEM((LANE,), _f32),
            pltpu.VMEM((LANE, OUT), _f32),
            pltpu.VMEM((LANE, OUT), _f32),
        ],
    )(x, ta, tb, W1, degp)


def _edge_pipeline(g_hbm, src_hbm, dst_hbm, acc_sp,
                   sidx_v, didx_v, rows_v, gsems, ssems, base, nchunks):
    def FG(g, h):
        for b in range(G):
            pltpu.async_copy(g_hbm.at[sidx_v.at[g * G + b]],
                             rows_v.at[h, b], gsems[h])

    def WG(g, h):
        for b in range(G):
            pltpu.make_async_copy(g_hbm.at[sidx_v.at[g * G + b]],
                                  rows_v.at[h, b], gsems[h]).wait()

    def FS(g, h):
        for b in range(G):
            pltpu.async_copy(rows_v.at[h, b],
                             acc_sp.at[didx_v.at[g * G + b]], ssems[h],
                             add=True)

    def WS(g, h):
        for b in range(G):
            pltpu.make_async_copy(rows_v.at[h, b],
                                  acc_sp.at[didx_v.at[g * G + b]],
                                  ssems[h]).wait()

    @pl.loop(0, nchunks)
    def _chunk(k):
        row0 = base + k * CH
        pltpu.sync_copy(src_hbm.at[pl.ds(row0, CH)], sidx_v)
        pltpu.sync_copy(dst_hbm.at[pl.ds(row0, CH)], didx_v)
        FG(0, 0)
        FG(1, 1)

        @pl.loop(0, NGRP // 2 - 1)
        def _pair(p):
            g0 = 2 * p
            WG(g0, 0)
            FS(g0, 0)
            WG(g0 + 1, 1)
            FS(g0 + 1, 1)
            WS(g0, 0)
            FG(g0 + 2, 0)
            WS(g0 + 1, 1)
            FG(g0 + 3, 1)

        WG(NGRP - 2, 0)
        FS(NGRP - 2, 0)
        WG(NGRP - 1, 1)
        FS(NGRP - 1, 1)
        WS(NGRP - 2, 0)
        WS(NGRP - 1, 1)


def _sc_scatter1_body(src_hbm, dst_hbm, ga_hbm, gb_hbm, z_hbm, out_hbm,
                      sidx_v, didx_v, rows_v, gsem0, gsem1, ssem0, ssem1,
                      acc_sp):
    c = lax.axis_index("c")
    s = lax.axis_index("s")

    pltpu.sync_copy(z_hbm, acc_sp.at[pl.ds(s * NPT, NPT)])
    plsc.subcore_barrier()

    args = (src_hbm, dst_hbm, acc_sp, sidx_v, didx_v, rows_v,
            (gsem0, gsem1), (ssem0, ssem1))

    @pl.when(c == 0)
    def _half_a():
        _edge_pipeline(ga_hbm, *args,
                       base=s * TILE_E_ROWS2, nchunks=TILE_E_ROWS2 // CH)

    @pl.when(c == 1)
    def _half_b():
        _edge_pipeline(gb_hbm, *args,
                       base=s * TILE_E_ROWS2, nchunks=TILE_E_ROWS2 // CH)

    plsc.subcore_barrier()
    pltpu.sync_copy(acc_sp.at[pl.ds(s * NPT, NPT)],
                    out_hbm.at[c, pl.ds(s * NPT, NPT)])


def _sc_scatter1(src_r, dst_r, ga, gb):
    z = jnp.zeros((NPT, OUT), _f32)
    return pl.kernel(
        _sc_scatter1_body,
        out_type=jax.ShapeDtypeStruct((NC, N_PAD, OUT), _f32),
        mesh=_mesh,
        compiler_params=_sc_params,
        scratch_types=[
            pltpu.VMEM((CH, LANE), _i32),
            pltpu.VMEM((CH, LANE), _i32),
            pltpu.VMEM((2, G, LANE, OUT), _f32),
            pltpu.SemaphoreType.DMA,
            pltpu.SemaphoreType.DMA,
            pltpu.SemaphoreType.DMA,
            pltpu.SemaphoreType.DMA,
            pltpu.VMEM_SHARED((N_PAD, OUT), _f32),
        ],
    )(src_r, dst_r, ga, gb, z)


def _sc_scatter2_body(src_hbm, dst_hbm, g_hbm, z_hbm, out_hbm,
                      sidx_v, didx_v, rows_v, gsem0, gsem1, ssem0, ssem1,
                      acc_sp):
    c = lax.axis_index("c")
    s = lax.axis_index("s")

    pltpu.sync_copy(z_hbm, acc_sp.at[pl.ds(s * NPT, NPT)])
    plsc.subcore_barrier()

    _edge_pipeline(g_hbm, src_hbm, dst_hbm, acc_sp, sidx_v, didx_v, rows_v,
                   (gsem0, gsem1), (ssem0, ssem1),
                   base=(c * NS + s) * TILE_E_ROWS,
                   nchunks=TILE_E_ROWS // CH)

    plsc.subcore_barrier()
    pltpu.sync_copy(acc_sp.at[pl.ds(s * NPT, NPT)],
                    out_hbm.at[c, pl.ds(s * NPT, NPT)])


def _sc_scatter2(src_r, dst_r, g):
    z = jnp.zeros((NPT, OUT), _f32)
    return pl.kernel(
        _sc_scatter2_body,
        out_type=jax.ShapeDtypeStruct((NC, N_PAD, OUT), _f32),
        mesh=_mesh,
        compiler_params=_sc_params,
        scratch_types=[
            pltpu.VMEM((CH, LANE), _i32),
            pltpu.VMEM((CH, LANE), _i32),
            pltpu.VMEM((2, G, LANE, OUT), _f32),
            pltpu.SemaphoreType.DMA,
            pltpu.SemaphoreType.DMA,
            pltpu.SemaphoreType.DMA,
            pltpu.SemaphoreType.DMA,
            pltpu.VMEM_SHARED((N_PAD, OUT), _f32),
        ],
    )(src_r, dst_r, g, z)


def _sc_node2_body(s1_hbm, g1a_hbm, g1b_hbm, dis_hbm, b1_hbm, W2_hbm,
                   g2_hbm,
                   sa_v, sb_v, ga_v, gb_v, dis_v, o1_v, w2_v, b1_v, g2_v):
    c = lax.axis_index("c")
    s = lax.axis_index("s")
    w = s * NC + c

    pltpu.sync_copy(W2_hbm, w2_v)
    pltpu.sync_copy(b1_hbm, b1_v)
    b1a = b1_v[pl.ds(0, 16)]
    b1b = b1_v[pl.ds(16, 16)]
    zero = jnp.zeros((16,), _f32)
    iota = lax.iota(_i32, 16)

    @pl.loop(w * NV // NW, (w + 1) * NV // NW)
    def _row(r):
        node0 = jnp.minimum(r * LANE, N - LANE)
        sl = pl.ds(node0, LANE)
        pltpu.sync_copy(s1_hbm.at[0, sl], sa_v)
        pltpu.sync_copy(s1_hbm.at[1, sl], sb_v)
        pltpu.sync_copy(g1a_hbm.at[sl], ga_v)
        pltpu.sync_copy(g1b_hbm.at[sl], gb_v)
        pltpu.sync_copy(dis_hbm.at[sl], dis_v)

        @pl.loop(0, LANE // 16)
        def _grp(t):
            base16 = t * 16
            dis16 = dis_v[pl.ds(base16, 16)]
            for j in range(16):
                n = base16 + j
                dn = dis16[j]
                o1a = jnp.maximum((sa_v[n, :] + ga_v[n, :]) * dn + b1a, zero)
                o1b = jnp.maximum((sb_v[n, :] + gb_v[n, :]) * dn + b1b, zero)
                acc = zero
                for k in range(OUT):
                    acc = acc + o1a[k] * w2_v[k, :]
                for k in range(OUT):
                    acc = acc + o1b[k] * w2_v[OUT + k, :]
                g2_v[n, :] = acc * dn

        pltpu.sync_copy(g2_v, g2_hbm.at[sl])


def _sc_node2(s1, g1a, g1b, dis, b1, W2):
    return pl.kernel(
        _sc_node2_body,
        out_type=jax.ShapeDtypeStruct((N_PAD, OUT), _f32),
        mesh=_mesh,
        compiler_params=_sc_params,
        scratch_types=[
            pltpu.VMEM((LANE, OUT), _f32),
            pltpu.VMEM((LANE, OUT), _f32),
            pltpu.VMEM((LANE, OUT), _f32),
            pltpu.VMEM((LANE, OUT), _f32),
            pltpu.VMEM((LANE,), _f32),
            pltpu.VMEM((HID, 16), _f32),
            pltpu.VMEM((HID, OUT), _f32),
            pltpu.VMEM((HID,), _f32),
            pltpu.VMEM((LANE, OUT), _f32),
        ],
    )(s1, g1a, g1b, dis, b1, W2)


def _sc_node3_body(s2_hbm, g2_hbm, dis_hbm, b2_hbm, out_hbm,
                   sa_v, sb_v, g2_v, dis_v, b2_v, o_v):
    c = lax.axis_index("c")
    s = lax.axis_index("s")
    w = s * NC + c

    pltpu.sync_copy(b2_hbm, b2_v)
    b2r = b2_v[...]
    zero = jnp.zeros((16,), _f32)
    iota = lax.iota(_i32, 16)

    @pl.loop(w * NV // NW, (w + 1) * NV // NW)
    def _row(r):
        node0 = jnp.minimum(r * LANE, N - LANE)
        sl = pl.ds(node0, LANE)
        pltpu.sync_copy(s2_hbm.at[0, sl], sa_v)
        pltpu.sync_copy(s2_hbm.at[1, sl], sb_v)
        pltpu.sync_copy(g2_hbm.at[sl], g2_v)
        pltpu.sync_copy(dis_hbm.at[sl], dis_v)

        @pl.loop(0, LANE // 16)
        def _grp(t):
            base16 = t * 16
            dis16 = dis_v[pl.ds(base16, 16)]
            for j in range(16):
                n = base16 + j
                o_v[n, :] = jnp.maximum(
                    (sa_v[n, :] + sb_v[n, :] + g2_v[n, :]) * dis16[j] + b2r,
                    zero)

        pltpu.sync_copy(o_v, out_hbm.at[sl])


def _sc_node3(s2, g2, dis, b2):
    return pl.kernel(
        _sc_node3_body,
        out_type=jax.ShapeDtypeStruct((N, OUT), _f32),
        mesh=_mesh,
        compiler_params=_sc_params,
        scratch_types=[
            pltpu.VMEM((LANE, OUT), _f32),
            pltpu.VMEM((LANE, OUT), _f32),
            pltpu.VMEM((LANE, OUT), _f32),
            pltpu.VMEM((LANE,), _f32),
            pltpu.VMEM((OUT,), _f32),
            pltpu.VMEM((LANE, OUT), _f32),
        ],
    )(s2, g2, dis, b2)


def kernel(x, edge_index, emb_a, emb_b, W1, b1, W2, b2):
    src = edge_index[0].astype(_i32)
    dst = edge_index[1].astype(_i32)
    pad_e = E_ROWS * LANE - E
    pad_src = jnp.zeros((pad_e,), _i32)
    pad_dst = N + (jnp.arange(pad_e, dtype=_i32) % (N_PAD - N))
    src_r = jnp.concatenate([src, pad_src]).reshape(E_ROWS, LANE)
    dst_r = jnp.concatenate([dst, pad_dst]).reshape(E_ROWS, LANE)

    ta, tb = _tc_tables(emb_a, emb_b, W1)
    degp = _sc_deg(dst_r).reshape(NC, N_PAD)
    dis, g1a, g1b = _sc_node1(x, ta, tb, W1, degp)
    s1 = _sc_scatter1(src_r, dst_r, g1a, g1b)
    g2 = _sc_node2(s1, g1a, g1b, dis, b1, W2)
    s2 = _sc_scatter2(src_r, dst_r, g2)
    return _sc_node3(s2, g2, dis, b2)
